# baseline (device time: 82305 ns/iter reference)
import jax
import jax.numpy as jnp
from jax import lax
from jax.experimental import pallas as pl
from jax.experimental.pallas import tpu as pltpu

N_DEV = 4
BF16 = jnp.bfloat16
C = 128


def kernel(x):
    m, n = x.shape
    h = m // 2
    q = m // 4
    e = m // 8
    K = e // C
    nslot = 12 * K

    def body(x_ref, out_ref, s1t, s1b, s2t, s2b, r1t, r1b, r2t, r2b,
             send_sems, recv_sems):
        my = lax.axis_index("i")
        xb = my // 2
        yb = (my % 2) ^ xb
        p1 = my ^ 1
        p2 = 3 - my

        qt_o = (1 - yb) * q
        qt = yb * q
        a_t = (1 - xb) * e
        b_t = xb * e
        g_t = qt + b_t
        o_t = qt + a_t

        qb_o = h + (1 - xb) * q
        qb = h + xb * q
        a_b = (1 - yb) * e
        b_b = yb * e
        g_b = qb + b_b
        o_b = qb + a_b

        barrier_sem = pltpu.get_barrier_semaphore()
        for nbr in (p1, p2):
            pl.semaphore_signal(
                barrier_sem, inc=1,
                device_id=(nbr,), device_id_type=pl.DeviceIdType.MESH,
            )
        pl.semaphore_wait(barrier_sem, 2)

        cs = {}
        slot_ctr = [0]

        def start(key, src, dst, peer):
            slot = slot_ctr[0]
            slot_ctr[0] += 1
            rdma = pltpu.make_async_remote_copy(
                src_ref=src, dst_ref=dst,
                send_sem=send_sems.at[slot], recv_sem=recv_sems.at[slot],
                device_id=(peer,), device_id_type=pl.DeviceIdType.MESH,
            )
            rdma.start()
            cs[key] = rdma

        def ck(ref, off, j):
            return ref.at[pl.ds(off + j * C, C), :]

        s1t[pl.ds(a_t, e), :] = x_ref[pl.ds(qt_o + a_t, e), :].astype(BF16)
        s1b[pl.ds(a_b, e), :] = x_ref[pl.ds(qb_o + a_b, e), :].astype(BF16)
        for j in range(K):
            start(("t1a", j), ck(s1t, a_t, j), ck(r1t, a_t, j), p1)
            start(("b1a", j), ck(s1b, a_b, j), ck(r1b, a_b, j), p2)
        s1t[pl.ds(b_t, e), :] = x_ref[pl.ds(qt_o + b_t, e), :].astype(BF16)
        s1b[pl.ds(b_b, e), :] = x_ref[pl.ds(qb_o + b_b, e), :].astype(BF16)

        for j in range(K):
            cs[("t1a", j)].wait_recv()
            s2t[pl.ds(j * C, C), :] = r1t[pl.ds(a_t + j * C, C), :] + x_ref[
                pl.ds(qt + a_t + j * C, C), :
            ].astype(BF16)
            start(("t2", j), ck(s2t, 0, j), ck(r2t, 0, j), p2)
            cs[("b1a", j)].wait_recv()
            s2b[pl.ds(j * C, C), :] = r1b[pl.ds(a_b + j * C, C), :] + x_ref[
                pl.ds(qb + a_b + j * C, C), :
            ].astype(BF16)
            start(("b2", j), ck(s2b, 0, j), ck(r2b, 0, j), p1)

        for j in range(K):
            start(("t1b", j), ck(s1t, b_t, j), ck(r1t, b_t, j), p1)
            start(("b1b", j), ck(s1b, b_b, j), ck(r1b, b_b, j), p2)

        for j in range(K):
            cs[("t2", j)].wait_recv()
            cs[("t1b", j)].wait_recv()
            out_ref[pl.ds(g_t + j * C, C), :] = (
                r2t[pl.ds(j * C, C), :] + r1t[pl.ds(b_t + j * C, C), :]
                + x_ref[pl.ds(g_t + j * C, C), :].astype(BF16)
            )
            start(("ag3t", j), ck(out_ref, g_t, j), ck(out_ref, g_t, j), p2)
            cs[("b2", j)].wait_recv()
            cs[("b1b", j)].wait_recv()
            out_ref[pl.ds(g_b + j * C, C), :] = (
                r2b[pl.ds(j * C, C), :] + r1b[pl.ds(b_b + j * C, C), :]
                + x_ref[pl.ds(g_b + j * C, C), :].astype(BF16)
            )
            start(("ag3b", j), ck(out_ref, g_b, j), ck(out_ref, g_b, j), p1)
            start(("ag4tm", j), ck(out_ref, g_t, j), ck(out_ref, g_t, j), p1)
            start(("ag4bm", j), ck(out_ref, g_b, j), ck(out_ref, g_b, j), p2)

        for j in range(K):
            cs[("ag3t", j)].wait_recv()
            start(("ag4to", j), ck(out_ref, o_t, j), ck(out_ref, o_t, j), p1)
            cs[("ag3b", j)].wait_recv()
            start(("ag4bo", j), ck(out_ref, o_b, j), ck(out_ref, o_b, j), p2)

        for j in range(K):
            for kind in ("ag4tm", "ag4bm", "ag4to", "ag4bo"):
                cs[(kind, j)].wait_recv()
        for c in cs.values():
            c.wait_send()

    return pl.pallas_call(
        body,
        out_shape=jax.ShapeDtypeStruct((m, n), BF16),
        in_specs=[pl.BlockSpec(memory_space=pltpu.VMEM)],
        out_specs=pl.BlockSpec(memory_space=pltpu.VMEM),
        scratch_shapes=[
            pltpu.VMEM((q, n), BF16),
            pltpu.VMEM((q, n), BF16),
            pltpu.VMEM((e, n), BF16),
            pltpu.VMEM((e, n), BF16),
            pltpu.VMEM((q, n), BF16),
            pltpu.VMEM((q, n), BF16),
            pltpu.VMEM((e, n), BF16),
            pltpu.VMEM((e, n), BF16),
            pltpu.SemaphoreType.DMA((nslot,)),
            pltpu.SemaphoreType.DMA((nslot,)),
        ],
        compiler_params=pltpu.CompilerParams(collective_id=0),
    )(x)


# device time: 81812 ns/iter; 1.0060x vs baseline; 1.0060x over previous
import jax
import jax.numpy as jnp
from jax import lax
from jax.experimental import pallas as pl
from jax.experimental.pallas import tpu as pltpu

N_DEV = 4
BF16 = jnp.bfloat16
C = 256


def kernel(x):
    m, n = x.shape
    h = m // 2
    q = m // 4
    e = m // 8
    K = e // C
    nslot = 12 * K

    def body(x_ref, out_ref, s1t, s1b, s2t, s2b, r1t, r1b, r2t, r2b,
             send_sems, recv_sems):
        my = lax.axis_index("i")
        xb = my // 2
        yb = (my % 2) ^ xb
        p1 = my ^ 1
        p2 = 3 - my

        qt_o = (1 - yb) * q
        qt = yb * q
        a_t = (1 - xb) * e
        b_t = xb * e
        g_t = qt + b_t
        o_t = qt + a_t

        qb_o = h + (1 - xb) * q
        qb = h + xb * q
        a_b = (1 - yb) * e
        b_b = yb * e
        g_b = qb + b_b
        o_b = qb + a_b

        barrier_sem = pltpu.get_barrier_semaphore()
        for nbr in (p1, p2):
            pl.semaphore_signal(
                barrier_sem, inc=1,
                device_id=(nbr,), device_id_type=pl.DeviceIdType.MESH,
            )
        pl.semaphore_wait(barrier_sem, 2)

        cs = {}
        slot_ctr = [0]

        def start(key, src, dst, peer):
            slot = slot_ctr[0]
            slot_ctr[0] += 1
            rdma = pltpu.make_async_remote_copy(
                src_ref=src, dst_ref=dst,
                send_sem=send_sems.at[slot], recv_sem=recv_sems.at[slot],
                device_id=(peer,), device_id_type=pl.DeviceIdType.MESH,
            )
            rdma.start()
            cs[key] = rdma

        def ck(ref, off, j):
            return ref.at[pl.ds(off + j * C, C), :]

        s1t[pl.ds(a_t, e), :] = x_ref[pl.ds(qt_o + a_t, e), :].astype(BF16)
        s1b[pl.ds(a_b, e), :] = x_ref[pl.ds(qb_o + a_b, e), :].astype(BF16)
        for j in range(K):
            start(("t1a", j), ck(s1t, a_t, j), ck(r1t, a_t, j), p1)
            start(("b1a", j), ck(s1b, a_b, j), ck(r1b, a_b, j), p2)
        s1t[pl.ds(b_t, e), :] = x_ref[pl.ds(qt_o + b_t, e), :].astype(BF16)
        s1b[pl.ds(b_b, e), :] = x_ref[pl.ds(qb_o + b_b, e), :].astype(BF16)

        for j in range(K):
            cs[("t1a", j)].wait_recv()
            s2t[pl.ds(j * C, C), :] = r1t[pl.ds(a_t + j * C, C), :] + x_ref[
                pl.ds(qt + a_t + j * C, C), :
            ].astype(BF16)
            start(("t2", j), ck(s2t, 0, j), ck(r2t, 0, j), p2)
            cs[("b1a", j)].wait_recv()
            s2b[pl.ds(j * C, C), :] = r1b[pl.ds(a_b + j * C, C), :] + x_ref[
                pl.ds(qb + a_b + j * C, C), :
            ].astype(BF16)
            start(("b2", j), ck(s2b, 0, j), ck(r2b, 0, j), p1)

        for j in range(K):
            start(("t1b", j), ck(s1t, b_t, j), ck(r1t, b_t, j), p1)
            start(("b1b", j), ck(s1b, b_b, j), ck(r1b, b_b, j), p2)

        for j in range(K):
            cs[("t2", j)].wait_recv()
            cs[("t1b", j)].wait_recv()
            out_ref[pl.ds(g_t + j * C, C), :] = (
                r2t[pl.ds(j * C, C), :] + r1t[pl.ds(b_t + j * C, C), :]
                + x_ref[pl.ds(g_t + j * C, C), :].astype(BF16)
            )
            start(("ag3t", j), ck(out_ref, g_t, j), ck(out_ref, g_t, j), p2)
            cs[("b2", j)].wait_recv()
            cs[("b1b", j)].wait_recv()
            out_ref[pl.ds(g_b + j * C, C), :] = (
                r2b[pl.ds(j * C, C), :] + r1b[pl.ds(b_b + j * C, C), :]
                + x_ref[pl.ds(g_b + j * C, C), :].astype(BF16)
            )
            start(("ag3b", j), ck(out_ref, g_b, j), ck(out_ref, g_b, j), p1)
            start(("ag4tm", j), ck(out_ref, g_t, j), ck(out_ref, g_t, j), p1)
            start(("ag4bm", j), ck(out_ref, g_b, j), ck(out_ref, g_b, j), p2)

        for j in range(K):
            cs[("ag3t", j)].wait_recv()
            start(("ag4to", j), ck(out_ref, o_t, j), ck(out_ref, o_t, j), p1)
            cs[("ag3b", j)].wait_recv()
            start(("ag4bo", j), ck(out_ref, o_b, j), ck(out_ref, o_b, j), p2)

        for j in range(K):
            for kind in ("ag4tm", "ag4bm", "ag4to", "ag4bo"):
                cs[(kind, j)].wait_recv()
        for c in cs.values():
            c.wait_send()

    return pl.pallas_call(
        body,
        out_shape=jax.ShapeDtypeStruct((m, n), BF16),
        in_specs=[pl.BlockSpec(memory_space=pltpu.VMEM)],
        out_specs=pl.BlockSpec(memory_space=pltpu.VMEM),
        scratch_shapes=[
            pltpu.VMEM((q, n), BF16),
            pltpu.VMEM((q, n), BF16),
            pltpu.VMEM((e, n), BF16),
            pltpu.VMEM((e, n), BF16),
            pltpu.VMEM((q, n), BF16),
            pltpu.VMEM((q, n), BF16),
            pltpu.VMEM((e, n), BF16),
            pltpu.VMEM((e, n), BF16),
            pltpu.SemaphoreType.DMA((nslot,)),
            pltpu.SemaphoreType.DMA((nslot,)),
        ],
        compiler_params=pltpu.CompilerParams(collective_id=0),
    )(x)
